# baseline (speedup 1.0000x reference)
import functools

import jax
import jax.numpy as jnp
from jax import lax
from jax.experimental import pallas as pl
from jax.experimental.pallas import tpu as pltpu
from jax.experimental.pallas import tpu_sc as plsc

N = 10000
E = 640000
F = 128
G = 128
NC = 2
NS = 16
NW = NC * NS
CH = 128
NCH = 160
EPW = NCH * CH
EPAD = NW * EPW - E
NB = 4
DEGW = 128
DEGP = 10240
DSTR = DEGP // NS
NPAD = 10240
RSTR = NPAD // NS

_mesh = plsc.VectorSubcoreMesh(core_axis_name="c", subcore_axis_name="s")


@functools.partial(
    pl.kernel,
    mesh=_mesh,
    out_type=jax.ShapeDtypeStruct((NC, DEGP, DEGW), jnp.float32),
    scratch_types=[
        pltpu.VMEM((CH,), jnp.int32),
        pltpu.VMEM((CH,), jnp.int32),
        pltpu.VMEM((CH, DEGW), jnp.float32),
        pltpu.VMEM_SHARED((DEGP, DEGW), jnp.float32),
        pltpu.SemaphoreType.DMA,
        pltpu.SemaphoreType.DMA,
    ],
)
def _deg_kernel(dst3, ones_hbm, zdeg, out, didx0, didx1, ones_v, acc, d0, d1):
    c = lax.axis_index("c")
    s = lax.axis_index("s")
    wid = s * NC + c
    dsem = (d0, d1)
    dbuf = (didx0, didx1)
    pltpu.sync_copy(ones_hbm, ones_v)
    for b in range(2):
        pltpu.async_copy(dst3.at[wid, b], dbuf[b], dsem[b])
    pltpu.sync_copy(zdeg.at[pl.ds(s * DSTR, DSTR)], acc.at[pl.ds(s * DSTR, DSTR)])
    plsc.subcore_barrier()

    def body(outer, _):
        for b in range(2):
            g_ = outer * 2 + b
            pltpu.make_async_copy(dst3.at[wid, 0], dbuf[b], dsem[b]).wait()
            pltpu.sync_copy(ones_v, acc.at[dbuf[b]], add=True)

            @pl.when(g_ + 2 < NCH)
            def _():
                pltpu.async_copy(dst3.at[wid, g_ + 2], dbuf[b], dsem[b])
        return 0

    lax.fori_loop(0, NCH // 2, body, 0)
    plsc.subcore_barrier()
    pltpu.sync_copy(acc.at[pl.ds(s * DSTR, DSTR)], out.at[c, pl.ds(s * DSTR, DSTR)])


@functools.partial(
    pl.kernel,
    mesh=_mesh,
    out_type=jax.ShapeDtypeStruct((NC, NPAD, F), jnp.float32),
    scratch_types=[
        pltpu.VMEM((2, CH), jnp.int32),
        pltpu.VMEM((2, CH), jnp.int32),
        pltpu.VMEM((2, CH, F), jnp.float32),
        pltpu.VMEM_SHARED((NPAD, F), jnp.float32),
        pltpu.SemaphoreType.DMA,
        pltpu.SemaphoreType.DMA,
        pltpu.SemaphoreType.DMA,
        pltpu.SemaphoreType.DMA,
        pltpu.SemaphoreType.DMA,
        pltpu.SemaphoreType.DMA,
    ],
)
def _agg_kernel(hp, src3, dst3, znode, out, sidx, didx, rows, acc,
                is0, is1, id0, id1, g0, g1):
    c = lax.axis_index("c")
    s = lax.axis_index("s")
    wid = s * NC + c
    isem = (is0, is1)
    dsem = (id0, id1)
    gsem = (g0, g1)
    pltpu.sync_copy(znode.at[pl.ds(s * RSTR, RSTR)], acc.at[pl.ds(s * RSTR, RSTR)])

    for b in range(2):
        pltpu.async_copy(src3.at[wid, b], sidx.at[b], isem[b])
        pltpu.async_copy(dst3.at[wid, b], didx.at[b], dsem[b])
    pltpu.make_async_copy(src3.at[wid, 0], sidx.at[0], isem[0]).wait()
    pltpu.async_copy(hp.at[sidx.at[0]], rows.at[0], gsem[0])
    plsc.subcore_barrier()

    def body(outer, _):
        for b in range(2):
            g_ = outer * 2 + b
            nb = 1 - b

            @pl.when(g_ + 1 < NCH)
            def _():
                pltpu.make_async_copy(src3.at[wid, 0], sidx.at[nb], isem[nb]).wait()
                pltpu.async_copy(hp.at[sidx.at[nb]], rows.at[nb], gsem[nb])

            pltpu.make_async_copy(hp.at[sidx.at[b]], rows.at[b], gsem[b]).wait()
            pltpu.make_async_copy(dst3.at[wid, 0], didx.at[b], dsem[b]).wait()
            pltpu.sync_copy(rows.at[b], acc.at[didx.at[b]], add=True)

            @pl.when(g_ + 2 < NCH)
            def _():
                pltpu.async_copy(src3.at[wid, g_ + 2], sidx.at[b], isem[b])
                pltpu.async_copy(dst3.at[wid, g_ + 2], didx.at[b], dsem[b])
        return 0

    lax.fori_loop(0, NCH // 2, body, 0)
    plsc.subcore_barrier()
    pltpu.sync_copy(acc.at[pl.ds(s * RSTR, RSTR)], out.at[c, pl.ds(s * RSTR, RSTR)])


ROWS_BLK = 2000


def _tc1_body(degp_ref, x_ref, w_ref, out_ref, dis_ref):
    deg = degp_ref[0, :, 0:1] + degp_ref[1, :, 0:1] + 1.0
    dis = lax.rsqrt(deg)
    h = jnp.dot(x_ref[...], w_ref[...], preferred_element_type=jnp.float32)
    out_ref[...] = h * dis
    dis_ref[...] = jnp.broadcast_to(dis, (ROWS_BLK, 16))


_tc1 = pl.pallas_call(
    _tc1_body,
    grid=(N // ROWS_BLK,),
    in_specs=[
        pl.BlockSpec((NC, ROWS_BLK, DEGW), lambda r: (0, r, 0)),
        pl.BlockSpec((ROWS_BLK, F), lambda r: (r, 0)),
        pl.BlockSpec((F, F), lambda r: (0, 0)),
    ],
    out_specs=[pl.BlockSpec((ROWS_BLK, F), lambda r: (r, 0)),
               pl.BlockSpec((ROWS_BLK, 16), lambda r: (r, 0))],
    out_shape=[jax.ShapeDtypeStruct((N, F), jnp.float32),
               jax.ShapeDtypeStruct((N, 16), jnp.float32)],
)


def _tcl_body(dis16_ref, p_ref, hp_ref, w_ref, b_ref, out_ref):
    dis = dis16_ref[:, 0:1]
    agg = p_ref[0] + p_ref[1] + hp_ref[...]
    xl = jnp.maximum(agg * dis + b_ref[...], 0.0)
    out_ref[...] = jnp.dot(xl, w_ref[...], preferred_element_type=jnp.float32) * dis


_tcl = pl.pallas_call(
    _tcl_body,
    grid=(N // ROWS_BLK,),
    in_specs=[
        pl.BlockSpec((ROWS_BLK, 16), lambda r: (r, 0)),
        pl.BlockSpec((NC, ROWS_BLK, F), lambda r: (0, r, 0)),
        pl.BlockSpec((ROWS_BLK, F), lambda r: (r, 0)),
        pl.BlockSpec((F, F), lambda r: (0, 0)),
        pl.BlockSpec((1, F), lambda r: (0, 0)),
    ],
    out_specs=pl.BlockSpec((ROWS_BLK, F), lambda r: (r, 0)),
    out_shape=jax.ShapeDtypeStruct((N, F), jnp.float32),
)


def _tcf_body(dis16_ref, p_ref, hp_ref, b_ref, out_ref):
    dis = dis16_ref[:, 0:1]
    agg = p_ref[0] + p_ref[1] + hp_ref[...]
    out_ref[...] = jnp.maximum(agg * dis + b_ref[...], 0.0)


_tcf = pl.pallas_call(
    _tcf_body,
    grid=(N // ROWS_BLK,),
    in_specs=[
        pl.BlockSpec((ROWS_BLK, 16), lambda r: (r, 0)),
        pl.BlockSpec((NC, ROWS_BLK, F), lambda r: (0, r, 0)),
        pl.BlockSpec((ROWS_BLK, F), lambda r: (r, 0)),
        pl.BlockSpec((1, F), lambda r: (0, 0)),
    ],
    out_specs=pl.BlockSpec((ROWS_BLK, F), lambda r: (r, 0)),
    out_shape=jax.ShapeDtypeStruct((N, F), jnp.float32),
)


def _pool_body(x4_ref, bt_ref, wl1_ref, bl1_ref, wl2_ref, bl2_ref, wl3_ref, bl3_ref, out_ref):
    bt = bt_ref[...]
    gids = lax.broadcasted_iota(jnp.int32, (1, G), 1)
    oh = (bt == gids).astype(jnp.float32)
    dn = (((0,), (0,)), ((), ()))
    sums = lax.dot_general(oh, x4_ref[...], dn, preferred_element_type=jnp.float32)
    cnts = lax.dot_general(oh, jnp.ones((N, 1), jnp.float32), dn,
                           preferred_element_type=jnp.float32)
    pooled = sums / jnp.maximum(cnts, 1.0)
    h = jnp.maximum(jnp.dot(pooled, wl1_ref[...], preferred_element_type=jnp.float32)
                    + bl1_ref[...], 0.0)
    h = jnp.maximum(jnp.dot(h, wl2_ref[...], preferred_element_type=jnp.float32)
                    + bl2_ref[...], 0.0)
    out_ref[...] = jnp.dot(h, wl3_ref[...], preferred_element_type=jnp.float32) + bl3_ref[...]


_pool = pl.pallas_call(
    _pool_body,
    out_shape=jax.ShapeDtypeStruct((G, 1), jnp.float32),
)


def kernel(x, edge_index, batch, W1, b1, W2, b2, W3, b3, Wl1, bl1, Wl2, bl2, Wl3, bl3):
    ones16 = jnp.ones((CH, DEGW), jnp.float32)
    zdeg = jnp.zeros((DEGP, DEGW), jnp.float32)
    znode = jnp.zeros((NPAD, F), jnp.float32)

    pad_src = jnp.zeros((EPAD,), jnp.int32)
    pad_dst = (jnp.arange(EPAD, dtype=jnp.int32) % (NPAD - N)) + N
    src3 = jnp.concatenate([edge_index[0], pad_src]).reshape(NW, NCH, CH)
    dst3 = jnp.concatenate([edge_index[1], pad_dst]).reshape(NW, NCH, CH)
    degp = _deg_kernel(dst3, ones16, zdeg)
    h1p, dis16 = _tc1(degp, x, W1)
    p1 = _agg_kernel(h1p, src3, dst3, znode)
    h2p = _tcl(dis16, p1, h1p, W2, b1.reshape(1, F))
    p2 = _agg_kernel(h2p, src3, dst3, znode)
    h3p = _tcl(dis16, p2, h2p, W3, b2.reshape(1, F))
    p3 = _agg_kernel(h3p, src3, dst3, znode)
    x4 = _tcf(dis16, p3, h3p, b3.reshape(1, F))
    outg = _pool(x4, batch.reshape(N, 1), Wl1, bl1.reshape(1, F),
                 Wl2, bl2.reshape(1, F), Wl3, bl3.reshape(1, 1))
    return outg.reshape(-1)


# speedup vs baseline: 3.8216x; 3.8216x over previous
import functools

import jax
import jax.numpy as jnp
from jax import lax
from jax.experimental import pallas as pl
from jax.experimental.pallas import tpu as pltpu
from jax.experimental.pallas import tpu_sc as plsc

N = 10000
E = 640000
F = 128
G = 128
NC = 2
NS = 16
NW = NC * NS
CH = 128
NCH = 158
EPW = NCH * CH
EPAD = NW * EPW - E
NREAL = E // CH
DEGW = 128
DEGP = 10240
DSTR = DEGP // NS
NPAD = 10240
RSTR = NPAD // NS

_mesh = plsc.VectorSubcoreMesh(core_axis_name="c", subcore_axis_name="s")


@functools.partial(
    pl.kernel,
    mesh=_mesh,
    out_type=jax.ShapeDtypeStruct((NC, DEGP, DEGW), jnp.float32),
    scratch_types=[
        pltpu.VMEM((CH,), jnp.int32),
        pltpu.VMEM((CH,), jnp.int32),
        pltpu.VMEM((CH, DEGW), jnp.float32),
        pltpu.VMEM_SHARED((DEGP, DEGW), jnp.float32),
        pltpu.SemaphoreType.DMA,
        pltpu.SemaphoreType.DMA,
    ],
)
def _deg_kernel(dst3, ones_hbm, zdeg, out, didx0, didx1, ones_v, acc, d0, d1):
    c = lax.axis_index("c")
    s = lax.axis_index("s")
    wid = s * NC + c
    dsem = (d0, d1)
    dbuf = (didx0, didx1)
    pltpu.sync_copy(ones_hbm, ones_v)
    for b in range(2):
        pltpu.async_copy(dst3.at[wid, b], dbuf[b], dsem[b])
    pltpu.sync_copy(zdeg.at[pl.ds(s * DSTR, DSTR)], acc.at[pl.ds(s * DSTR, DSTR)])
    plsc.subcore_barrier()

    def body(outer, _):
        for b in range(2):
            g_ = outer * 2 + b
            pltpu.make_async_copy(dst3.at[wid, 0], dbuf[b], dsem[b]).wait()

            @pl.when(wid * NCH + g_ < NREAL)
            def _():
                pltpu.sync_copy(ones_v, acc.at[dbuf[b]], add=True)

            @pl.when(g_ + 2 < NCH)
            def _():
                pltpu.async_copy(dst3.at[wid, g_ + 2], dbuf[b], dsem[b])
        return 0

    lax.fori_loop(0, NCH // 2, body, 0)
    plsc.subcore_barrier()
    pltpu.sync_copy(acc.at[pl.ds(s * DSTR, DSTR)], out.at[c, pl.ds(s * DSTR, DSTR)])


@functools.partial(
    pl.kernel,
    mesh=_mesh,
    out_type=jax.ShapeDtypeStruct((NC, NPAD, F), jnp.float32),
    scratch_types=[
        pltpu.VMEM((2, CH), jnp.int32),
        pltpu.VMEM((2, CH), jnp.int32),
        pltpu.VMEM((2, CH, F), jnp.float32),
        pltpu.VMEM_SHARED((NPAD, F), jnp.float32),
        pltpu.SemaphoreType.DMA,
        pltpu.SemaphoreType.DMA,
        pltpu.SemaphoreType.DMA,
        pltpu.SemaphoreType.DMA,
        pltpu.SemaphoreType.DMA,
        pltpu.SemaphoreType.DMA,
    ],
)
def _agg_kernel(hp, src3, dst3, znode, out, sidx, didx, rows, acc,
                is0, is1, id0, id1, g0, g1):
    c = lax.axis_index("c")
    s = lax.axis_index("s")
    wid = s * NC + c
    isem = (is0, is1)
    dsem = (id0, id1)
    gsem = (g0, g1)
    pltpu.sync_copy(znode.at[pl.ds(s * RSTR, RSTR)], acc.at[pl.ds(s * RSTR, RSTR)])

    for b in range(2):
        pltpu.async_copy(src3.at[wid, b], sidx.at[b], isem[b])
        pltpu.async_copy(dst3.at[wid, b], didx.at[b], dsem[b])
    pltpu.make_async_copy(src3.at[wid, 0], sidx.at[0], isem[0]).wait()
    pltpu.async_copy(hp.at[sidx.at[0]], rows.at[0], gsem[0])
    plsc.subcore_barrier()

    def body(outer, _):
        for b in range(2):
            g_ = outer * 2 + b
            nb = 1 - b

            @pl.when((g_ + 1 < NCH) & (wid * NCH + g_ + 1 < NREAL))
            def _():
                pltpu.make_async_copy(src3.at[wid, 0], sidx.at[nb], isem[nb]).wait()
                pltpu.async_copy(hp.at[sidx.at[nb]], rows.at[nb], gsem[nb])

            @pl.when(wid * NCH + g_ < NREAL)
            def _():
                pltpu.make_async_copy(hp.at[sidx.at[b]], rows.at[b], gsem[b]).wait()
                pltpu.make_async_copy(dst3.at[wid, 0], didx.at[b], dsem[b]).wait()
                pltpu.sync_copy(rows.at[b], acc.at[didx.at[b]], add=True)

            @pl.when((g_ + 2 < NCH) & (wid * NCH + g_ + 2 < NREAL))
            def _():
                pltpu.async_copy(src3.at[wid, g_ + 2], sidx.at[b], isem[b])
                pltpu.async_copy(dst3.at[wid, g_ + 2], didx.at[b], dsem[b])
        return 0

    lax.fori_loop(0, NCH // 2, body, 0)
    plsc.subcore_barrier()
    pltpu.sync_copy(acc.at[pl.ds(s * RSTR, RSTR)], out.at[c, pl.ds(s * RSTR, RSTR)])


ROWS_BLK = 2000


def _tc1_body(degp_ref, x_ref, w_ref, out_ref, dis_ref):
    deg = degp_ref[0, :, 0:1] + degp_ref[1, :, 0:1] + 1.0
    dis = lax.rsqrt(deg)
    h = jnp.dot(x_ref[...], w_ref[...], preferred_element_type=jnp.float32)
    out_ref[...] = h * dis
    dis_ref[...] = jnp.broadcast_to(dis, (ROWS_BLK, 16))


_tc1 = pl.pallas_call(
    _tc1_body,
    grid=(N // ROWS_BLK,),
    in_specs=[
        pl.BlockSpec((NC, ROWS_BLK, DEGW), lambda r: (0, r, 0)),
        pl.BlockSpec((ROWS_BLK, F), lambda r: (r, 0)),
        pl.BlockSpec((F, F), lambda r: (0, 0)),
    ],
    out_specs=[pl.BlockSpec((ROWS_BLK, F), lambda r: (r, 0)),
               pl.BlockSpec((ROWS_BLK, 16), lambda r: (r, 0))],
    out_shape=[jax.ShapeDtypeStruct((N, F), jnp.float32),
               jax.ShapeDtypeStruct((N, 16), jnp.float32)],
)


def _tcl_body(dis16_ref, p_ref, hp_ref, w_ref, b_ref, out_ref):
    dis = dis16_ref[:, 0:1]
    agg = p_ref[0] + p_ref[1] + hp_ref[...]
    xl = jnp.maximum(agg * dis + b_ref[...], 0.0)
    out_ref[...] = jnp.dot(xl, w_ref[...], preferred_element_type=jnp.float32) * dis


_tcl = pl.pallas_call(
    _tcl_body,
    grid=(N // ROWS_BLK,),
    in_specs=[
        pl.BlockSpec((ROWS_BLK, 16), lambda r: (r, 0)),
        pl.BlockSpec((NC, ROWS_BLK, F), lambda r: (0, r, 0)),
        pl.BlockSpec((ROWS_BLK, F), lambda r: (r, 0)),
        pl.BlockSpec((F, F), lambda r: (0, 0)),
        pl.BlockSpec((1, F), lambda r: (0, 0)),
    ],
    out_specs=pl.BlockSpec((ROWS_BLK, F), lambda r: (r, 0)),
    out_shape=jax.ShapeDtypeStruct((N, F), jnp.float32),
)


def _tcf_body(dis16_ref, p_ref, hp_ref, b_ref, out_ref):
    dis = dis16_ref[:, 0:1]
    agg = p_ref[0] + p_ref[1] + hp_ref[...]
    out_ref[...] = jnp.maximum(agg * dis + b_ref[...], 0.0)


_tcf = pl.pallas_call(
    _tcf_body,
    grid=(N // ROWS_BLK,),
    in_specs=[
        pl.BlockSpec((ROWS_BLK, 16), lambda r: (r, 0)),
        pl.BlockSpec((NC, ROWS_BLK, F), lambda r: (0, r, 0)),
        pl.BlockSpec((ROWS_BLK, F), lambda r: (r, 0)),
        pl.BlockSpec((1, F), lambda r: (0, 0)),
    ],
    out_specs=pl.BlockSpec((ROWS_BLK, F), lambda r: (r, 0)),
    out_shape=jax.ShapeDtypeStruct((N, F), jnp.float32),
)


def _pool_body(x4_ref, bt_ref, wl1_ref, bl1_ref, wl2_ref, bl2_ref, wl3_ref, bl3_ref, out_ref):
    bt = bt_ref[...]
    gids = lax.broadcasted_iota(jnp.int32, (1, G), 1)
    oh = (bt == gids).astype(jnp.float32)
    dn = (((0,), (0,)), ((), ()))
    sums = lax.dot_general(oh, x4_ref[...], dn, preferred_element_type=jnp.float32)
    cnts = lax.dot_general(oh, jnp.ones((N, 1), jnp.float32), dn,
                           preferred_element_type=jnp.float32)
    pooled = sums / jnp.maximum(cnts, 1.0)
    h = jnp.maximum(jnp.dot(pooled, wl1_ref[...], preferred_element_type=jnp.float32)
                    + bl1_ref[...], 0.0)
    h = jnp.maximum(jnp.dot(h, wl2_ref[...], preferred_element_type=jnp.float32)
                    + bl2_ref[...], 0.0)
    out_ref[...] = jnp.dot(h, wl3_ref[...], preferred_element_type=jnp.float32) + bl3_ref[...]


_pool = pl.pallas_call(
    _pool_body,
    out_shape=jax.ShapeDtypeStruct((G, 1), jnp.float32),
)


def kernel(x, edge_index, batch, W1, b1, W2, b2, W3, b3, Wl1, bl1, Wl2, bl2, Wl3, bl3):
    ones16 = jnp.ones((CH, DEGW), jnp.float32)
    zdeg = jnp.zeros((DEGP, DEGW), jnp.float32)
    znode = jnp.zeros((NPAD, F), jnp.float32)

    pad_src = jnp.zeros((EPAD,), jnp.int32)
    pad_dst = (jnp.arange(EPAD, dtype=jnp.int32) % (NPAD - N)) + N
    src3 = jnp.concatenate([edge_index[0], pad_src]).reshape(NW, NCH, CH)
    dst3 = jnp.concatenate([edge_index[1], pad_dst]).reshape(NW, NCH, CH)
    degp = _deg_kernel(dst3, ones16, zdeg)
    h1p, dis16 = _tc1(degp, x, W1)
    p1 = _agg_kernel(h1p, src3, dst3, znode)
    h2p = _tcl(dis16, p1, h1p, W2, b1.reshape(1, F))
    p2 = _agg_kernel(h2p, src3, dst3, znode)
    h3p = _tcl(dis16, p2, h2p, W3, b2.reshape(1, F))
    p3 = _agg_kernel(h3p, src3, dst3, znode)
    x4 = _tcf(dis16, p3, h3p, b3.reshape(1, F))
    outg = _pool(x4, batch.reshape(N, 1), Wl1, bl1.reshape(1, F),
                 Wl2, bl2.reshape(1, F), Wl3, bl3.reshape(1, 1))
    return outg.reshape(-1)


# speedup vs baseline: 4.1350x; 1.0820x over previous
import functools

import jax
import jax.numpy as jnp
from jax import lax
from jax.experimental import pallas as pl
from jax.experimental.pallas import tpu as pltpu
from jax.experimental.pallas import tpu_sc as plsc

N = 10000
E = 640000
F = 128
G = 128
NC = 2
NS = 16
NW = NC * NS
CH = 128
NCH = 158
EPW = NCH * CH
EPAD = NW * EPW - E
NREAL = E // CH
NPAD = 10240
RSTR = NPAD // NS
HL = 8
HSZ = NPAD * HL

_mesh = plsc.VectorSubcoreMesh(core_axis_name="c", subcore_axis_name="s")


@functools.partial(
    pl.kernel,
    mesh=_mesh,
    out_type=jax.ShapeDtypeStruct((NW, HSZ), jnp.float32),
    scratch_types=[
        pltpu.VMEM((2, CH), jnp.int32),
        pltpu.VMEM((HSZ,), jnp.float32),
        pltpu.SemaphoreType.DMA,
        pltpu.SemaphoreType.DMA,
    ],
    compiler_params=pltpu.CompilerParams(needs_layout_passes=False),
)
def _deg_kernel(dst3, out, didx, hist, d0, d1):
    c = lax.axis_index("c")
    s = lax.axis_index("s")
    wid = s * NC + c
    dsem = (d0, d1)
    for b in range(2):
        pltpu.async_copy(dst3.at[wid, b], didx.at[b], dsem[b])
    zero16 = jnp.zeros((16,), jnp.float32)

    def zbody(i, _):
        for j in range(8):
            hist[pl.ds(i * 128 + j * 16, 16)] = zero16
        return 0

    lax.fori_loop(0, HSZ // 128, zbody, 0)
    ones16v = jnp.ones((16,), jnp.float32)
    lane8 = lax.bitwise_and(lax.iota(jnp.int32, 16), 7)

    def body(outer, _):
        for b in range(2):
            g_ = outer * 2 + b
            pltpu.make_async_copy(dst3.at[wid, 0], didx.at[b], dsem[b]).wait()

            @pl.when(wid * NCH + g_ < NREAL)
            def _():
                for j in range(8):
                    iv = didx[b, pl.ds(j * 16, 16)]
                    hidx = lax.shift_left(iv, 3) + lane8
                    plsc.addupdate_scatter(hist, [hidx], ones16v)

            @pl.when(g_ + 2 < NCH)
            def _():
                pltpu.async_copy(dst3.at[wid, g_ + 2], didx.at[b], dsem[b])
        return 0

    lax.fori_loop(0, NCH // 2, body, 0)
    pltpu.sync_copy(hist, out.at[wid])


@functools.partial(
    pl.kernel,
    mesh=_mesh,
    out_type=jax.ShapeDtypeStruct((NC, NPAD, F), jnp.float32),
    scratch_types=[
        pltpu.VMEM((2, CH), jnp.int32),
        pltpu.VMEM((2, CH), jnp.int32),
        pltpu.VMEM((2, CH, F), jnp.float32),
        pltpu.VMEM_SHARED((NPAD, F), jnp.float32),
        pltpu.SemaphoreType.DMA,
        pltpu.SemaphoreType.DMA,
        pltpu.SemaphoreType.DMA,
        pltpu.SemaphoreType.DMA,
        pltpu.SemaphoreType.DMA,
        pltpu.SemaphoreType.DMA,
    ],
)
def _agg_kernel(hp, src3, dst3, znode, out, sidx, didx, rows, acc,
                is0, is1, id0, id1, g0, g1):
    c = lax.axis_index("c")
    s = lax.axis_index("s")
    wid = s * NC + c
    isem = (is0, is1)
    dsem = (id0, id1)
    gsem = (g0, g1)
    pltpu.sync_copy(znode.at[pl.ds(s * RSTR, RSTR)], acc.at[pl.ds(s * RSTR, RSTR)])

    for b in range(2):
        pltpu.async_copy(src3.at[wid, b], sidx.at[b], isem[b])
        pltpu.async_copy(dst3.at[wid, b], didx.at[b], dsem[b])
    pltpu.make_async_copy(src3.at[wid, 0], sidx.at[0], isem[0]).wait()
    pltpu.async_copy(hp.at[sidx.at[0]], rows.at[0], gsem[0])
    plsc.subcore_barrier()

    def body(outer, _):
        for b in range(2):
            g_ = outer * 2 + b
            nb = 1 - b

            @pl.when((g_ + 1 < NCH) & (wid * NCH + g_ + 1 < NREAL))
            def _():
                pltpu.make_async_copy(src3.at[wid, 0], sidx.at[nb], isem[nb]).wait()
                pltpu.async_copy(hp.at[sidx.at[nb]], rows.at[nb], gsem[nb])

            @pl.when(wid * NCH + g_ < NREAL)
            def _():
                pltpu.make_async_copy(hp.at[sidx.at[b]], rows.at[b], gsem[b]).wait()
                pltpu.make_async_copy(dst3.at[wid, 0], didx.at[b], dsem[b]).wait()
                pltpu.sync_copy(rows.at[b], acc.at[didx.at[b]], add=True)

            @pl.when((g_ + 2 < NCH) & (wid * NCH + g_ + 2 < NREAL))
            def _():
                pltpu.async_copy(src3.at[wid, g_ + 2], sidx.at[b], isem[b])
                pltpu.async_copy(dst3.at[wid, g_ + 2], didx.at[b], dsem[b])
        return 0

    lax.fori_loop(0, NCH // 2, body, 0)
    plsc.subcore_barrier()
    pltpu.sync_copy(acc.at[pl.ds(s * RSTR, RSTR)], out.at[c, pl.ds(s * RSTR, RSTR)])


ROWS_BLK = 2000


def _tcsum_body(degp_ref, out_ref):
    out_ref[...] = jnp.sum(degp_ref[...], axis=0, keepdims=True)


_TSB = 16384
_tcsum = pl.pallas_call(
    _tcsum_body,
    grid=(NPAD * HL // _TSB,),
    in_specs=[pl.BlockSpec((NW, _TSB), lambda r: (0, r))],
    out_specs=pl.BlockSpec((1, _TSB), lambda r: (0, r)),
    out_shape=jax.ShapeDtypeStruct((1, NPAD * HL), jnp.float32),
)


def _tc1_body(degp_ref, x_ref, w_ref, out_ref, dis_ref):
    deg = jnp.sum(degp_ref[...], axis=1, keepdims=True) + 1.0
    dis = lax.rsqrt(deg)
    h = jnp.dot(x_ref[...], w_ref[...], preferred_element_type=jnp.float32)
    out_ref[...] = h * dis
    dis_ref[...] = jnp.broadcast_to(dis, (ROWS_BLK, 16))


_tc1 = pl.pallas_call(
    _tc1_body,
    grid=(N // ROWS_BLK,),
    in_specs=[
        pl.BlockSpec((ROWS_BLK, HL), lambda r: (r, 0)),
        pl.BlockSpec((ROWS_BLK, F), lambda r: (r, 0)),
        pl.BlockSpec((F, F), lambda r: (0, 0)),
    ],
    out_specs=[pl.BlockSpec((ROWS_BLK, F), lambda r: (r, 0)),
               pl.BlockSpec((ROWS_BLK, 16), lambda r: (r, 0))],
    out_shape=[jax.ShapeDtypeStruct((N, F), jnp.float32),
               jax.ShapeDtypeStruct((N, 16), jnp.float32)],
)


def _tcl_body(dis16_ref, p_ref, hp_ref, w_ref, b_ref, out_ref):
    dis = dis16_ref[:, 0:1]
    agg = p_ref[0] + p_ref[1] + hp_ref[...]
    xl = jnp.maximum(agg * dis + b_ref[...], 0.0)
    out_ref[...] = jnp.dot(xl, w_ref[...], preferred_element_type=jnp.float32) * dis


_tcl = pl.pallas_call(
    _tcl_body,
    grid=(N // ROWS_BLK,),
    in_specs=[
        pl.BlockSpec((ROWS_BLK, 16), lambda r: (r, 0)),
        pl.BlockSpec((NC, ROWS_BLK, F), lambda r: (0, r, 0)),
        pl.BlockSpec((ROWS_BLK, F), lambda r: (r, 0)),
        pl.BlockSpec((F, F), lambda r: (0, 0)),
        pl.BlockSpec((1, F), lambda r: (0, 0)),
    ],
    out_specs=pl.BlockSpec((ROWS_BLK, F), lambda r: (r, 0)),
    out_shape=jax.ShapeDtypeStruct((N, F), jnp.float32),
)


def _tcf_body(dis16_ref, p_ref, hp_ref, b_ref, out_ref):
    dis = dis16_ref[:, 0:1]
    agg = p_ref[0] + p_ref[1] + hp_ref[...]
    out_ref[...] = jnp.maximum(agg * dis + b_ref[...], 0.0)


_tcf = pl.pallas_call(
    _tcf_body,
    grid=(N // ROWS_BLK,),
    in_specs=[
        pl.BlockSpec((ROWS_BLK, 16), lambda r: (r, 0)),
        pl.BlockSpec((NC, ROWS_BLK, F), lambda r: (0, r, 0)),
        pl.BlockSpec((ROWS_BLK, F), lambda r: (r, 0)),
        pl.BlockSpec((1, F), lambda r: (0, 0)),
    ],
    out_specs=pl.BlockSpec((ROWS_BLK, F), lambda r: (r, 0)),
    out_shape=jax.ShapeDtypeStruct((N, F), jnp.float32),
)


def _pool_body(x4_ref, bt_ref, wl1_ref, bl1_ref, wl2_ref, bl2_ref, wl3_ref, bl3_ref, out_ref):
    bt = bt_ref[...]
    gids = lax.broadcasted_iota(jnp.int32, (1, G), 1)
    oh = (bt == gids).astype(jnp.float32)
    dn = (((0,), (0,)), ((), ()))
    sums = lax.dot_general(oh, x4_ref[...], dn, preferred_element_type=jnp.float32)
    cnts = lax.dot_general(oh, jnp.ones((N, 1), jnp.float32), dn,
                           preferred_element_type=jnp.float32)
    pooled = sums / jnp.maximum(cnts, 1.0)
    h = jnp.maximum(jnp.dot(pooled, wl1_ref[...], preferred_element_type=jnp.float32)
                    + bl1_ref[...], 0.0)
    h = jnp.maximum(jnp.dot(h, wl2_ref[...], preferred_element_type=jnp.float32)
                    + bl2_ref[...], 0.0)
    out_ref[...] = jnp.dot(h, wl3_ref[...], preferred_element_type=jnp.float32) + bl3_ref[...]


_pool = pl.pallas_call(
    _pool_body,
    out_shape=jax.ShapeDtypeStruct((G, 1), jnp.float32),
)


def kernel(x, edge_index, batch, W1, b1, W2, b2, W3, b3, Wl1, bl1, Wl2, bl2, Wl3, bl3):
    znode = jnp.zeros((NPAD, F), jnp.float32)

    pad_src = jnp.zeros((EPAD,), jnp.int32)
    pad_dst = (jnp.arange(EPAD, dtype=jnp.int32) % (NPAD - N)) + N
    src3 = jnp.concatenate([edge_index[0], pad_src]).reshape(NW, NCH, CH)
    dst3 = jnp.concatenate([edge_index[1], pad_dst]).reshape(NW, NCH, CH)
    degp1d = _deg_kernel(dst3)
    deg8 = _tcsum(degp1d).reshape(NPAD, HL)
    h1p, dis16 = _tc1(deg8, x, W1)
    p1 = _agg_kernel(h1p, src3, dst3, znode)
    h2p = _tcl(dis16, p1, h1p, W2, b1.reshape(1, F))
    p2 = _agg_kernel(h2p, src3, dst3, znode)
    h3p = _tcl(dis16, p2, h2p, W3, b2.reshape(1, F))
    p3 = _agg_kernel(h3p, src3, dst3, znode)
    x4 = _tcf(dis16, p3, h3p, b3.reshape(1, F))
    outg = _pool(x4, batch.reshape(N, 1), Wl1, bl1.reshape(1, F),
                 Wl2, bl2.reshape(1, F), Wl3, bl3.reshape(1, 1))
    return outg.reshape(-1)


# speedup vs baseline: 4.6552x; 1.1258x over previous
import functools

import jax
import jax.numpy as jnp
from jax import lax
from jax.experimental import pallas as pl
from jax.experimental.pallas import tpu as pltpu
from jax.experimental.pallas import tpu_sc as plsc

N = 10000
E = 640000
F = 128
G = 128
NC = 2
NS = 16
NW = NC * NS
CH = 128
NCH = 159
EPW = NCH * CH
EPAD = NW * EPW - E
NREAL = E // CH
NPAD = 10112
RSTR = NPAD // NS
HL = 8
HSZ = NPAD * HL

_mesh = plsc.VectorSubcoreMesh(core_axis_name="c", subcore_axis_name="s")


@functools.partial(
    pl.kernel,
    mesh=_mesh,
    out_type=jax.ShapeDtypeStruct((NW, HSZ), jnp.float32),
    scratch_types=[
        pltpu.VMEM((2, CH), jnp.int32),
        pltpu.VMEM((HSZ,), jnp.float32),
        pltpu.SemaphoreType.DMA,
        pltpu.SemaphoreType.DMA,
    ],
    compiler_params=pltpu.CompilerParams(needs_layout_passes=False),
)
def _deg_kernel(dst3, out, didx, hist, d0, d1):
    c = lax.axis_index("c")
    s = lax.axis_index("s")
    wid = s * NC + c
    dsem = (d0, d1)
    for b in range(2):
        pltpu.async_copy(dst3.at[wid, b], didx.at[b], dsem[b])
    zero16 = jnp.zeros((16,), jnp.float32)

    def zbody(i, _):
        for j in range(8):
            hist[pl.ds(i * 128 + j * 16, 16)] = zero16
        return 0

    lax.fori_loop(0, HSZ // 128, zbody, 0)
    ones16v = jnp.ones((16,), jnp.float32)
    lane8 = lax.bitwise_and(lax.iota(jnp.int32, 16), 7)

    def body(outer, _):
        for b in range(2):
            g_ = outer * 2 + b
            pltpu.make_async_copy(dst3.at[wid, 0], didx.at[b], dsem[b]).wait()

            @pl.when(wid * NCH + g_ < NREAL)
            def _():
                for j in range(8):
                    iv = didx[b, pl.ds(j * 16, 16)]
                    hidx = lax.shift_left(iv, 3) + lane8
                    plsc.addupdate_scatter(hist, [hidx], ones16v)

            @pl.when(g_ + 2 < NCH)
            def _():
                pltpu.async_copy(dst3.at[wid, g_ + 2], didx.at[b], dsem[b])
        return 0

    lax.fori_loop(0, NCH // 2, body, 0)
    if NCH % 2:
        g_last = NCH - 1
        pltpu.make_async_copy(dst3.at[wid, 0], didx.at[0], d0).wait()

        @pl.when(wid * NCH + g_last < NREAL)
        def _():
            for j in range(8):
                iv = didx[0, pl.ds(j * 16, 16)]
                hidx = lax.shift_left(iv, 3) + lane8
                plsc.addupdate_scatter(hist, [hidx], ones16v)

    pltpu.sync_copy(hist, out.at[wid])


@functools.partial(
    pl.kernel,
    mesh=_mesh,
    out_type=jax.ShapeDtypeStruct((NC, NPAD, F), jnp.float32),
    scratch_types=[
        pltpu.VMEM((3, CH), jnp.int32),
        pltpu.VMEM((3, CH), jnp.int32),
        pltpu.VMEM((3, CH, F), jnp.float32),
        pltpu.VMEM_SHARED((NPAD, F), jnp.float32),
        pltpu.SemaphoreType.DMA,
        pltpu.SemaphoreType.DMA,
        pltpu.SemaphoreType.DMA,
        pltpu.SemaphoreType.DMA,
        pltpu.SemaphoreType.DMA,
        pltpu.SemaphoreType.DMA,
        pltpu.SemaphoreType.DMA,
        pltpu.SemaphoreType.DMA,
        pltpu.SemaphoreType.DMA,
    ],
)
def _agg_kernel(hp, src3, dst3, znode, out, sidx, didx, rows, acc,
                is0, is1, is2, id0, id1, id2, g0, g1, g2):
    c = lax.axis_index("c")
    s = lax.axis_index("s")
    wid = s * NC + c
    isem = (is0, is1, is2)
    dsem = (id0, id1, id2)
    gsem = (g0, g1, g2)
    pltpu.sync_copy(znode.at[pl.ds(s * RSTR, RSTR)], acc.at[pl.ds(s * RSTR, RSTR)])

    for b in range(3):
        pltpu.async_copy(src3.at[wid, b], sidx.at[b], isem[b])
        pltpu.async_copy(dst3.at[wid, b], didx.at[b], dsem[b])
    pltpu.make_async_copy(src3.at[wid, 0], sidx.at[0], isem[0]).wait()
    pltpu.async_copy(hp.at[sidx.at[0]], rows.at[0], gsem[0])
    plsc.subcore_barrier()

    def body(outer, _):
        for b in range(3):
            g_ = outer * 3 + b
            nb = (b + 1) % 3

            @pl.when((g_ + 1 < NCH) & (wid * NCH + g_ + 1 < NREAL))
            def _():
                pltpu.make_async_copy(src3.at[wid, 0], sidx.at[nb], isem[nb]).wait()
                pltpu.async_copy(hp.at[sidx.at[nb]], rows.at[nb], gsem[nb])

            @pl.when(wid * NCH + g_ < NREAL)
            def _():
                pltpu.make_async_copy(hp.at[sidx.at[b]], rows.at[b], gsem[b]).wait()
                pltpu.make_async_copy(dst3.at[wid, 0], didx.at[b], dsem[b]).wait()
                pltpu.sync_copy(rows.at[b], acc.at[didx.at[b]], add=True)

            @pl.when((g_ + 3 < NCH) & (wid * NCH + g_ + 3 < NREAL))
            def _():
                pltpu.async_copy(src3.at[wid, g_ + 3], sidx.at[b], isem[b])
                pltpu.async_copy(dst3.at[wid, g_ + 3], didx.at[b], dsem[b])
        return 0

    lax.fori_loop(0, NCH // 3, body, 0)
    plsc.subcore_barrier()
    pltpu.sync_copy(acc.at[pl.ds(s * RSTR, RSTR)], out.at[c, pl.ds(s * RSTR, RSTR)])


ROWS_BLK = 2000


def _tcsum_body(degp_ref, out_ref):
    out_ref[...] = jnp.sum(degp_ref[...], axis=0, keepdims=True)


_TSB = NPAD * HL // 4
_tcsum = pl.pallas_call(
    _tcsum_body,
    grid=(NPAD * HL // _TSB,),
    in_specs=[pl.BlockSpec((NW, _TSB), lambda r: (0, r))],
    out_specs=pl.BlockSpec((1, _TSB), lambda r: (0, r)),
    out_shape=jax.ShapeDtypeStruct((1, NPAD * HL), jnp.float32),
)


def _tc1_body(degp_ref, x_ref, w_ref, out_ref, dis_ref):
    deg = jnp.sum(degp_ref[...], axis=1, keepdims=True) + 1.0
    dis = lax.rsqrt(deg)
    h = jnp.dot(x_ref[...], w_ref[...], preferred_element_type=jnp.float32)
    out_ref[...] = h * dis
    dis_ref[...] = jnp.broadcast_to(dis, (ROWS_BLK, 16))


_tc1 = pl.pallas_call(
    _tc1_body,
    grid=(N // ROWS_BLK,),
    in_specs=[
        pl.BlockSpec((ROWS_BLK, HL), lambda r: (r, 0)),
        pl.BlockSpec((ROWS_BLK, F), lambda r: (r, 0)),
        pl.BlockSpec((F, F), lambda r: (0, 0)),
    ],
    out_specs=[pl.BlockSpec((ROWS_BLK, F), lambda r: (r, 0)),
               pl.BlockSpec((ROWS_BLK, 16), lambda r: (r, 0))],
    out_shape=[jax.ShapeDtypeStruct((N, F), jnp.float32),
               jax.ShapeDtypeStruct((N, 16), jnp.float32)],
)


def _tcl_body(dis16_ref, p_ref, hp_ref, w_ref, b_ref, out_ref):
    dis = dis16_ref[:, 0:1]
    agg = p_ref[0] + p_ref[1] + hp_ref[...]
    xl = jnp.maximum(agg * dis + b_ref[...], 0.0)
    out_ref[...] = jnp.dot(xl, w_ref[...], preferred_element_type=jnp.float32) * dis


_tcl = pl.pallas_call(
    _tcl_body,
    grid=(N // ROWS_BLK,),
    in_specs=[
        pl.BlockSpec((ROWS_BLK, 16), lambda r: (r, 0)),
        pl.BlockSpec((NC, ROWS_BLK, F), lambda r: (0, r, 0)),
        pl.BlockSpec((ROWS_BLK, F), lambda r: (r, 0)),
        pl.BlockSpec((F, F), lambda r: (0, 0)),
        pl.BlockSpec((1, F), lambda r: (0, 0)),
    ],
    out_specs=pl.BlockSpec((ROWS_BLK, F), lambda r: (r, 0)),
    out_shape=jax.ShapeDtypeStruct((N, F), jnp.float32),
)


def _tcf_body(dis16_ref, p_ref, hp_ref, b_ref, out_ref):
    dis = dis16_ref[:, 0:1]
    agg = p_ref[0] + p_ref[1] + hp_ref[...]
    out_ref[...] = jnp.maximum(agg * dis + b_ref[...], 0.0)


_tcf = pl.pallas_call(
    _tcf_body,
    grid=(N // ROWS_BLK,),
    in_specs=[
        pl.BlockSpec((ROWS_BLK, 16), lambda r: (r, 0)),
        pl.BlockSpec((NC, ROWS_BLK, F), lambda r: (0, r, 0)),
        pl.BlockSpec((ROWS_BLK, F), lambda r: (r, 0)),
        pl.BlockSpec((1, F), lambda r: (0, 0)),
    ],
    out_specs=pl.BlockSpec((ROWS_BLK, F), lambda r: (r, 0)),
    out_shape=jax.ShapeDtypeStruct((N, F), jnp.float32),
)


def _pool_body(x4_ref, bt_ref, wl1_ref, bl1_ref, wl2_ref, bl2_ref, wl3_ref, bl3_ref, out_ref):
    bt = bt_ref[...]
    gids = lax.broadcasted_iota(jnp.int32, (1, G), 1)
    oh = (bt == gids).astype(jnp.float32)
    dn = (((0,), (0,)), ((), ()))
    sums = lax.dot_general(oh, x4_ref[...], dn, preferred_element_type=jnp.float32)
    cnts = lax.dot_general(oh, jnp.ones((N, 1), jnp.float32), dn,
                           preferred_element_type=jnp.float32)
    pooled = sums / jnp.maximum(cnts, 1.0)
    h = jnp.maximum(jnp.dot(pooled, wl1_ref[...], preferred_element_type=jnp.float32)
                    + bl1_ref[...], 0.0)
    h = jnp.maximum(jnp.dot(h, wl2_ref[...], preferred_element_type=jnp.float32)
                    + bl2_ref[...], 0.0)
    out_ref[...] = jnp.dot(h, wl3_ref[...], preferred_element_type=jnp.float32) + bl3_ref[...]


_pool = pl.pallas_call(
    _pool_body,
    out_shape=jax.ShapeDtypeStruct((G, 1), jnp.float32),
)


def kernel(x, edge_index, batch, W1, b1, W2, b2, W3, b3, Wl1, bl1, Wl2, bl2, Wl3, bl3):
    znode = jnp.zeros((NPAD, F), jnp.float32)

    pad_src = jnp.zeros((EPAD,), jnp.int32)
    pad_dst = (jnp.arange(EPAD, dtype=jnp.int32) % (NPAD - N)) + N
    src3 = jnp.concatenate([edge_index[0], pad_src]).reshape(NW, NCH, CH)
    dst3 = jnp.concatenate([edge_index[1], pad_dst]).reshape(NW, NCH, CH)
    degp1d = _deg_kernel(dst3)
    deg8 = _tcsum(degp1d).reshape(NPAD, HL)
    h1p, dis16 = _tc1(deg8, x, W1)
    p1 = _agg_kernel(h1p, src3, dst3, znode)
    h2p = _tcl(dis16, p1, h1p, W2, b1.reshape(1, F))
    p2 = _agg_kernel(h2p, src3, dst3, znode)
    h3p = _tcl(dis16, p2, h2p, W3, b2.reshape(1, F))
    p3 = _agg_kernel(h3p, src3, dst3, znode)
    x4 = _tcf(dis16, p3, h3p, b3.reshape(1, F))
    outg = _pool(x4, batch.reshape(N, 1), Wl1, bl1.reshape(1, F),
                 Wl2, bl2.reshape(1, F), Wl3, bl3.reshape(1, 1))
    return outg.reshape(-1)
